# Initial kernel scaffold; baseline (speedup 1.0000x reference)
import jax
import jax.numpy as jnp
from jax.experimental import pallas as pl


def kernel(unary, binary, index1, index2, w_unary, w_binary):
    raise NotImplementedError("write your pallas kernel here")


# speedup vs baseline: 20.4875x; 20.4875x over previous
import functools

import jax
import jax.numpy as jnp
from jax import lax
from jax.experimental import pallas as pl
from jax.experimental.pallas import tpu as pltpu
from jax.experimental.pallas import tpu_sc as plsc

N_NODES = 50000
N_UNARY = 16
N_EDGES = 800000

NC, NS, L = 2, 16, 16
NW = NC * NS

NPW = 1664
NPAD = NW * NPW
EPW = 25600
EPAD = NW * EPW
RCH = 208
ECH = 1024
NGC = ECH // L

_mesh = plsc.VectorSubcoreMesh(core_axis_name="c", subcore_axis_name="s")


def _wid():
    return lax.axis_index("s") * NC + lax.axis_index("c")


def _f32(shape):
    return jax.ShapeDtypeStruct(shape, jnp.float32)


def _i32(shape):
    return jax.ShapeDtypeStruct(shape, jnp.int32)


def _row(ref2d, r, iot):
    return plsc.load_gather(ref2d, [jnp.full((L,), r, jnp.int32), iot])


@functools.partial(
    pl.kernel,
    out_type=(
        _f32((NPAD * L,)),
        _f32((NPAD * L,)),
        _f32((NPAD * L,)),
        _f32((EPAD,)),
        _i32((NW * NPAD,)),
        _i32((NW * NPAD,)),
    ),
    mesh=_mesh,
    compiler_params=pltpu.CompilerParams(needs_layout_passes=False, use_tc_tiling_on_sc=False),
    scratch_types=[
        pltpu.VMEM((RCH * L,), jnp.float32),
        pltpu.VMEM((RCH * L,), jnp.float32),
        pltpu.VMEM((RCH * L,), jnp.float32),
        pltpu.VMEM((ECH,), jnp.float32),
        pltpu.VMEM((ECH,), jnp.float32),
        pltpu.VMEM((ECH,), jnp.int32),
        pltpu.VMEM((ECH,), jnp.int32),
        pltpu.VMEM((NPAD,), jnp.int32),
        pltpu.VMEM((NPAD,), jnp.int32),
        pltpu.VMEM((L,), jnp.float32),
    ],
)
def _k1(unary_h, binary_h, i1_h, i2_h, coef_h,
        u_h, x_h, y_h, eb_h, l1_h, l2_h,
        ubuf, xbuf, ybuf, bv, ebv, i1v, i2v, l1v, l2v, cfv):
    w = _wid()
    iot = lax.iota(jnp.int32, L)
    pltpu.sync_copy(coef_h, cfv)
    coef = cfv[...]
    sign = jnp.where((iot & 1) == 1, 1.0, -1.0).astype(jnp.float32)
    perm = iot ^ 1

    rbase = w * NPW

    def chunk_a(ci, _):
        r0 = (rbase + ci * RCH) * L
        pltpu.sync_copy(unary_h.at[pl.ds(r0, RCH * L)], ubuf)

        def row(r, _):
            sl = pl.ds(r * L, L)
            x = ubuf[sl]
            xs = plsc.load_gather(ubuf, [r * L + perm])
            t = sign * (x + xs)
            sg = 1.0 / (1.0 + jnp.exp(-t))
            u = x + coef * sg
            ubuf[sl] = u
            xbuf[sl] = jnp.exp(-u)
            ybuf[sl] = jnp.exp(u)
            return 0

        lax.fori_loop(0, RCH, row, 0)
        pltpu.sync_copy(ubuf, u_h.at[pl.ds(r0, RCH * L)])
        pltpu.sync_copy(xbuf, x_h.at[pl.ds(r0, RCH * L)])
        pltpu.sync_copy(ybuf, y_h.at[pl.ds(r0, RCH * L)])
        return 0

    lax.fori_loop(0, NPW // RCH, chunk_a, 0)

    ebase = w * EPW

    def chunk_b(ci, _):
        e0 = ebase + ci * ECH
        pltpu.sync_copy(binary_h.at[pl.ds(e0, ECH)], bv)

        def grp(gi, _):
            sl = pl.ds(gi * L, L)
            ebv[sl] = jnp.exp(-bv[sl])
            return 0

        lax.fori_loop(0, NGC, grp, 0)
        pltpu.sync_copy(ebv, eb_h.at[pl.ds(e0, ECH)])
        return 0

    lax.fori_loop(0, EPW // ECH, chunk_b, 0)

    neg1 = jnp.full((L,), -1, jnp.int32)

    def init(i, _):
        l1v[pl.ds(i * L, L)] = neg1
        l2v[pl.ds(i * L, L)] = neg1
        return 0

    lax.fori_loop(0, NPAD // L, init, 0)

    def chunk_c(ci, _):
        e0 = ebase + ci * ECH
        pltpu.sync_copy(i1_h.at[pl.ds(e0, ECH)], i1v)
        pltpu.sync_copy(i2_h.at[pl.ds(e0, ECH)], i2v)

        def grp(gi, _):
            ev = e0 + gi * L + iot
            valid = ev < N_EDGES
            n1 = i1v[pl.ds(gi * L, L)]
            n2 = i2v[pl.ds(gi * L, L)]

            def upd(tbl, nv):
                def cond(p):
                    return jnp.any(p)

                def body(p):
                    plsc.store_scatter(tbl, [nv], ev, mask=p)
                    got = plsc.load_gather(tbl, [nv])
                    return valid & (got < ev)

                lax.while_loop(cond, body, valid)

            upd(l1v, n1)
            upd(l2v, n2)
            return 0

        lax.fori_loop(0, NGC, grp, 0)
        return 0

    lax.fori_loop(0, EPW // ECH, chunk_c, 0)
    pltpu.sync_copy(l1v, l1_h.at[pl.ds(w * NPAD, NPAD)])
    pltpu.sync_copy(l2v, l2_h.at[pl.ds(w * NPAD, NPAD)])


@functools.partial(
    pl.kernel,
    out_type=_f32((EPAD,)),
    mesh=_mesh,
    compiler_params=pltpu.CompilerParams(needs_layout_passes=False, use_tc_tiling_on_sc=False),
    scratch_types=[
        pltpu.VMEM((ECH,), jnp.int32),
        pltpu.VMEM((ECH,), jnp.int32),
        pltpu.VMEM((ECH,), jnp.float32),
        pltpu.VMEM((ECH,), jnp.float32),
        pltpu.VMEM((ECH, L), jnp.float32),
        pltpu.VMEM((ECH, L), jnp.float32),
        pltpu.VMEM((ECH,), jnp.float32),
        pltpu.VMEM((L,), jnp.float32),
        pltpu.SemaphoreType.DMA,
        pltpu.SemaphoreType.DMA,
    ],
)
def _k2(x_h, y_h, eb_h, binary_h, i1_h, i2_h, wb_h,
        bo_h, i1v, i2v, bv, ebv, r1v, r2v, bov, wbv, sem1, sem2):
    w = _wid()
    ebase = w * EPW
    iot = lax.iota(jnp.int32, L)
    pltpu.sync_copy(wb_h, wbv)
    wb = wbv[...]

    def chunk(ci, _):
        e0 = ebase + ci * ECH
        pltpu.sync_copy(i1_h.at[pl.ds(e0, ECH)], i1v)
        pltpu.sync_copy(i2_h.at[pl.ds(e0, ECH)], i2v)
        pltpu.sync_copy(binary_h.at[pl.ds(e0, ECH)], bv)
        pltpu.sync_copy(eb_h.at[pl.ds(e0, ECH)], ebv)
        for j in range(ECH // 128):
            c1 = pltpu.async_copy(
                x_h.at[i1v.at[pl.ds(j * 128, 128)]],
                r1v.at[pl.ds(j * 128, 128)], sem1)
            c2 = pltpu.async_copy(
                y_h.at[i2v.at[pl.ds(j * 128, 128)]],
                r2v.at[pl.ds(j * 128, 128)], sem2)
            c1.wait()
            c2.wait()

        def grp(gi, _):
            sl = pl.ds(gi * L, L)
            ebg = ebv[sl]
            sacc = jnp.zeros((L,), jnp.float32)
            for j in range(L):
                e = gi * L + j
                x = _row(r1v, e, iot)
                y = _row(r2v, e, iot)
                d = x + y + ebg[j]
                s = jnp.sum(wb / d)
                sacc = jnp.where(iot == j, s, sacc)
            bov[sl] = bv[sl] - ebg * sacc
            return 0

        lax.fori_loop(0, NGC, grp, 0)
        pltpu.sync_copy(bov, bo_h.at[pl.ds(e0, ECH)])
        return 0

    lax.fori_loop(0, EPW // ECH, chunk, 0)


_TSL = 128


@functools.partial(
    pl.kernel,
    out_type=_f32((NPAD * L,)),
    mesh=_mesh,
    compiler_params=pltpu.CompilerParams(needs_layout_passes=False, use_tc_tiling_on_sc=False),
    scratch_types=[
        pltpu.VMEM((NPW * L,), jnp.float32),
        pltpu.VMEM((8 * NPW,), jnp.int32),
        pltpu.VMEM((NPW,), jnp.int32),
        pltpu.VMEM((NPW,), jnp.int32),
        pltpu.VMEM((NPW,), jnp.int32),
        pltpu.VMEM((NPW,), jnp.int32),
        pltpu.VMEM((NPW,), jnp.int32),
        pltpu.VMEM((NPW,), jnp.int32),
        pltpu.VMEM((NPW,), jnp.float32),
        pltpu.VMEM((NPW,), jnp.float32),
        pltpu.VMEM((NPW, L), jnp.float32),
        pltpu.VMEM((NPW, L), jnp.float32),
        pltpu.VMEM((L,), jnp.float32),
        pltpu.SemaphoreType.DMA,
        pltpu.SemaphoreType.DMA,
        pltpu.SemaphoreType.DMA,
        pltpu.SemaphoreType.DMA,
    ],
)
def _k3(u_h, x_h, y_h, eb_h, i1_h, i2_h, l1_h, l2_h, wb_h,
        uo_h, uv, tb, acc1, acc2, e1v, e2v, i2e, i1e, eb1, eb2,
        yg, xg, wbv, s1, s2, s3, s4):
    w = _wid()
    rb = w * NPW
    iot = lax.iota(jnp.int32, L)
    pltpu.sync_copy(wb_h, wbv)
    wb = wbv[...]
    pltpu.sync_copy(u_h.at[pl.ds(rb * L, NPW * L)], uv)

    neg1 = jnp.full((L,), -1, jnp.int32)

    def init(i, _):
        acc1[pl.ds(i * L, L)] = neg1
        acc2[pl.ds(i * L, L)] = neg1
        return 0

    lax.fori_loop(0, NPW // L, init, 0)

    for tbl_h, acc in ((l1_h, acc1), (l2_h, acc2)):
        for rnd in range(NW // 8):
            handles = []
            for b in range(8):
                handles.append(pltpu.async_copy(
                    tbl_h.at[pl.ds((rnd * 8 + b) * NPAD + rb, NPW)],
                    tb.at[pl.ds(b * NPW, NPW)], s1))
            for h in handles:
                h.wait()

            def mrg(gi, _):
                sl = pl.ds(gi * L, L)
                a = acc[sl]
                for b in range(8):
                    a = jnp.maximum(a, tb[pl.ds(b * NPW + gi * L, L)])
                acc[sl] = a
                return 0

            lax.fori_loop(0, NPW // L, mrg, 0)

    def clamp(gi, _):
        sl = pl.ds(gi * L, L)
        e1v[sl] = jnp.maximum(acc1[sl], 0)
        e2v[sl] = jnp.maximum(acc2[sl], 0)
        return 0

    lax.fori_loop(0, NPW // L, clamp, 0)

    for j in range(NPW // _TSL):
        sl = pl.ds(j * _TSL, _TSL)
        h1 = pltpu.async_copy(i2_h.at[e1v.at[sl]], i2e.at[sl], s1)
        h2 = pltpu.async_copy(eb_h.at[e1v.at[sl]], eb1.at[sl], s2)
        h3 = pltpu.async_copy(i1_h.at[e2v.at[sl]], i1e.at[sl], s3)
        h4 = pltpu.async_copy(eb_h.at[e2v.at[sl]], eb2.at[sl], s4)
        h1.wait()
        h2.wait()
        h3.wait()
        h4.wait()

    for j in range(NPW // _TSL):
        sl = pl.ds(j * _TSL, _TSL)
        h1 = pltpu.async_copy(y_h.at[i2e.at[sl]], yg.at[sl], s1)
        h2 = pltpu.async_copy(x_h.at[i1e.at[sl]], xg.at[sl], s2)
        h1.wait()
        h2.wait()

    def grp(gi, _):
        sl = pl.ds(gi * L, L)
        a1 = acc1[sl]
        a2 = acc2[sl]
        e1g = eb1[sl]
        e2g = eb2[sl]
        for j in range(L):
            r = gi * L + j
            rsl = pl.ds(r * L, L)
            u = uv[rsl]
            xo = jnp.exp(-u)
            yo = jnp.exp(u)
            d1 = xo + _row(yg, r, iot) + e1g[j]
            c1 = jnp.where(a1[j] >= 0, -(wb * xo) / d1, 0.0)
            d2 = _row(xg, r, iot) + yo + e2g[j]
            c2 = jnp.where(a2[j] >= 0, (wb * yo) / d2, 0.0)
            uv[rsl] = u + c1 + c2
        return 0

    lax.fori_loop(0, NPW // L, grp, 0)
    pltpu.sync_copy(uv, uo_h.at[pl.ds(rb * L, NPW * L)])


def kernel(unary, binary, index1, index2, w_unary, w_binary):
    unary_p = jnp.pad(unary, ((0, NPAD - N_NODES), (0, 0)))
    binary_p = jnp.pad(binary, (0, EPAD - N_EDGES))
    pad_idx = (jnp.arange(EPAD - N_EDGES, dtype=jnp.int32) * 64) % N_NODES
    i1p = jnp.concatenate([index1, pad_idx])
    i2p = jnp.concatenate([index2, pad_idx])
    lanes = jnp.arange(L)
    sgn = jnp.where(lanes % 2 == 1, 1.0, -1.0)
    wrep = jnp.repeat(w_unary, 2, total_repeat_length=8)
    coef = jnp.where(lanes < 8, sgn * jnp.pad(wrep, (0, 8)), 0.0)
    coef = coef.astype(jnp.float32)

    u, x, y, eb, l1, l2 = _k1(unary_p.reshape(-1), binary_p, i1p, i2p, coef)
    x2 = x.reshape(NPAD, L)
    y2 = y.reshape(NPAD, L)
    bo = _k2(x2, y2, eb, binary_p, i1p, i2p, w_binary)
    uo = _k3(u, x2, y2, eb, i1p, i2p, l1, l2, w_binary)
    return (uo.reshape(NPAD, L)[:N_NODES], bo[:N_EDGES])
